# Initial kernel scaffold; baseline (speedup 1.0000x reference)
import jax
import jax.numpy as jnp
from jax.experimental import pallas as pl


def kernel(features, adjacency_matrices, W_rel, a_rel, bias):
    raise NotImplementedError("write your pallas kernel here")


# speedup vs baseline: 8.3201x; 8.3201x over previous
import jax
import jax.numpy as jnp
from jax.experimental import pallas as pl
from jax.experimental.pallas import tpu as pltpu

N = 2048
DIN = 256
DOUT = 256
R = 4
H = 4
DH = DOUT // H
BM = 256
NB = N // BM
BW = 384


def _rgat_kernel(feat_ref, adj_ref, w_ref, asrc_ref, adst_ref, bias_ref,
                 out_ref, b_scr, es_scr):
    i = pl.program_id(0)
    r = pl.program_id(1)

    row256 = jax.lax.broadcasted_iota(jnp.int32, (DOUT, 128), 0)
    col128 = jax.lax.broadcasted_iota(jnp.int32, (DOUT, 128), 1)
    sum_mat = (row256 // DH == col128).astype(jnp.float32)
    row128 = jax.lax.broadcasted_iota(jnp.int32, (128, DOUT), 0)
    col256 = jax.lax.broadcasted_iota(jnp.int32, (128, DOUT), 1)
    g2 = (row128 == col256 // DH).astype(jnp.float32)

    @pl.when(i == 0)
    def _prep():
        feat = feat_ref[...]
        w = w_ref[0]
        t = jnp.dot(feat, w.T, preferred_element_type=jnp.float32)
        adst = adst_ref[0]
        asrc = asrc_ref[0]
        ssd = jnp.dot(t * adst, sum_mat, preferred_element_type=jnp.float32)
        sdb = jnp.dot(ssd, g2, preferred_element_type=jnp.float32)
        ev = jnp.exp(sdb)
        b_scr[r, :, 0:DOUT] = ev * t
        ccol = jax.lax.broadcasted_iota(jnp.int32, (N, 128), 1)
        vcols = jnp.where(ccol < H, jnp.exp(ssd), 0.0) \
            + jnp.where(ccol == H, 1.0, 0.0)
        b_scr[r, :, DOUT:BW] = vcols
        sss = jnp.dot(t * asrc, sum_mat, preferred_element_type=jnp.float32)
        es_scr[r, :, :] = jnp.exp(sss)

    a_blk = adj_ref[0].astype(jnp.float32)
    p = jnp.dot(a_blk, b_scr[r], preferred_element_type=jnp.float32)
    m = p[:, 0:DOUT]
    s1b = jnp.dot(p[:, DOUT:BW], g2, preferred_element_type=jnp.float32)
    deg = p[:, DOUT + H:DOUT + H + 1]
    es_blk = es_scr[r, pl.ds(i * BM, BM), :]
    esb = jnp.dot(es_blk, g2, preferred_element_type=jnp.float32)
    z = esb * s1b + (jnp.float32(N) - deg)
    contrib = (esb * m / z) * jnp.float32(1.0 / R)

    @pl.when(r == 0)
    def _init():
        out_ref[...] = contrib + bias_ref[...]

    @pl.when(r > 0)
    def _acc():
        out_ref[...] += contrib


def kernel(features, adjacency_matrices, W_rel, a_rel, bias):
    asrc_t = jnp.tile(a_rel[:, :DH], (1, H)).reshape(R, 1, DOUT)
    adst_t = jnp.tile(a_rel[:, DH:], (1, H)).reshape(R, 1, DOUT)
    bias2d = bias.reshape(1, DOUT)

    grid = (NB, R)
    out = pl.pallas_call(
        _rgat_kernel,
        grid=grid,
        in_specs=[
            pl.BlockSpec((N, DIN), lambda i, r: (0, 0)),
            pl.BlockSpec((1, BM, N), lambda i, r: (r, i, 0)),
            pl.BlockSpec((1, DOUT, DIN), lambda i, r: (r, 0, 0)),
            pl.BlockSpec((1, 1, DOUT), lambda i, r: (r, 0, 0)),
            pl.BlockSpec((1, 1, DOUT), lambda i, r: (r, 0, 0)),
            pl.BlockSpec((1, DOUT), lambda i, r: (0, 0)),
        ],
        out_specs=pl.BlockSpec((BM, DOUT), lambda i, r: (i, 0)),
        out_shape=jax.ShapeDtypeStruct((N, DOUT), jnp.float32),
        scratch_shapes=[
            pltpu.VMEM((R, N, BW), jnp.float32),
            pltpu.VMEM((R, N, 128), jnp.float32),
        ],
        compiler_params=pltpu.CompilerParams(
            dimension_semantics=("arbitrary", "arbitrary"),
        ),
    )(features, adjacency_matrices, W_rel, asrc_t, adst_t, bias2d)
    return out
